# baseline (speedup 1.0000x reference)
import functools

import numpy as np

import jax
import jax.numpy as jnp
from jax import lax
from jax.experimental import pallas as pl
from jax.experimental.pallas import tpu as pltpu
from jax.experimental.pallas import tpu_sc as plsc

_LEVELS = [40962, 10242, 2562, 642, 162]
_EPS = 1e-5

_NC = 2
_NS = 16
_NW = _NC * _NS


def _rup(x, m):
    return (x + m - 1) // m * m


@functools.lru_cache(maxsize=None)
def _make_sc_gather(V, D, Bp):
    assert Bp % (8 * _NW) == 0
    b_per_w = Bp // _NW
    C = min(128 if D <= 256 else 64, b_per_w)
    NBUF = max(1, min(8, 393216 // (C * D * 4)))
    nfull = b_per_w // C
    tail = b_per_w % C
    ngrp = nfull // NBUF
    nrem = nfull % NBUF

    mesh = plsc.VectorSubcoreMesh(core_axis_name="c", subcore_axis_name="s")
    scratch = [pltpu.VMEM((b_per_w,), jnp.int32)]
    scratch += [pltpu.VMEM((C, D), jnp.float32) for _ in range(NBUF)]
    scratch += [pltpu.SemaphoreType.DMA for _ in range(2 * NBUF)]

    @functools.partial(
        pl.kernel,
        mesh=mesh,
        out_type=jax.ShapeDtypeStruct((Bp, D), jnp.float32),
        compiler_params=pltpu.CompilerParams(use_tc_tiling_on_sc=False),
        scratch_types=scratch,
    )
    def gather_kernel(table_hbm, idx_hbm, out_hbm, idx_v, *rest):
        bufs = rest[:NBUF]
        gsem = rest[NBUF : 2 * NBUF]
        wsem = rest[2 * NBUF : 3 * NBUF]
        wid = lax.axis_index("s") * _NC + lax.axis_index("c")
        base = wid * b_per_w
        pltpu.sync_copy(idx_hbm.at[pl.ds(base, b_per_w)], idx_v)

        def fire_gather(off, b):
            pltpu.async_copy(
                table_hbm.at[idx_v.at[pl.ds(off, C)]], bufs[b], gsem[b]
            )

        def wait_gather(b):
            pltpu.make_async_copy(
                table_hbm.at[idx_v.at[pl.ds(0, C)]], bufs[b], gsem[b]
            ).wait()

        def fire_wb(off, b):
            pltpu.async_copy(bufs[b], out_hbm.at[pl.ds(base + off, C)], wsem[b])

        def wait_wb(b):
            pltpu.make_async_copy(bufs[b], out_hbm.at[pl.ds(0, C)], wsem[b]).wait()

        def group(g, carry):
            for b in range(NBUF):
                c = g * NBUF + b

                @pl.when(g > 0)
                def _(b=b):
                    wait_wb(b)

                fire_gather(c * C, b)
                if b > 0:
                    wait_gather(b - 1)
                    fire_wb((c - 1) * C, b - 1)
                else:

                    @pl.when(g > 0)
                    def _(c=c):
                        wait_gather(NBUF - 1)
                        fire_wb((c - 1) * C, NBUF - 1)

            return carry

        if ngrp > 0:
            lax.fori_loop(0, ngrp, group, 0)
            wait_gather(NBUF - 1)
            fire_wb((ngrp * NBUF - 1) * C, NBUF - 1)
            for b in range(NBUF):
                wait_wb(b)

        off0 = ngrp * NBUF * C
        for j in range(nrem):
            off = off0 + j * C
            pltpu.async_copy(
                table_hbm.at[idx_v.at[pl.ds(off, C)]], bufs[0], gsem[0]
            ).wait()
            pltpu.sync_copy(bufs[0], out_hbm.at[pl.ds(base + off, C)])
        if tail:
            off = nfull * C
            pltpu.async_copy(
                table_hbm.at[idx_v.at[pl.ds(off, tail)]],
                bufs[0].at[pl.ds(0, tail)],
                gsem[0],
            ).wait()
            pltpu.sync_copy(
                bufs[0].at[pl.ds(0, tail)], out_hbm.at[pl.ds(base + off, tail)]
            )

    return gather_kernel


def _sc_gather(table, idxp):
    V, D = table.shape
    (Bp,) = idxp.shape
    return _make_sc_gather(V, D, Bp)(table, idxp)


def _tc_gather6(table, no6p, npad):
    V, f = table.shape
    idx2 = no6p.reshape(npad, 6)
    BN = _pick_bn(npad)

    def body(idx_ref, t_ref, out_ref):
        iota = lax.broadcasted_iota(jnp.int32, (BN, V), 1)
        t = t_ref[...]
        for k in range(6):
            sel = idx_ref[:, k : k + 1]
            M = (iota == sel).astype(jnp.float32)
            out_ref[:, k * f : (k + 1) * f] = jnp.dot(
                M, t, preferred_element_type=jnp.float32
            )

    return pl.pallas_call(
        body,
        grid=(npad // BN,),
        in_specs=[
            pl.BlockSpec((BN, 6), lambda i: (i, 0)),
            pl.BlockSpec((V, f), lambda i: (0, 0)),
        ],
        out_specs=pl.BlockSpec((BN, 6 * f), lambda i: (i, 0)),
        out_shape=jax.ShapeDtypeStruct((npad, 6 * f), jnp.float32),
    )(idx2, table)


def _tc_gather1(table, idxp):
    V, f = table.shape
    (Bp,) = idxp.shape
    BN = _pick_bn(Bp)

    def body(idx_ref, t_ref, out_ref):
        iota = lax.broadcasted_iota(jnp.int32, (BN, V), 1)
        M = (iota == idx_ref[...]).astype(jnp.float32)
        out_ref[...] = jnp.dot(M, t_ref[...], preferred_element_type=jnp.float32)

    return pl.pallas_call(
        body,
        grid=(Bp // BN,),
        in_specs=[
            pl.BlockSpec((BN, 1), lambda i: (i, 0)),
            pl.BlockSpec((V, f), lambda i: (0, 0)),
        ],
        out_specs=pl.BlockSpec((BN, f), lambda i: (i, 0)),
        out_shape=jax.ShapeDtypeStruct((Bp, f), jnp.float32),
    )(idxp.reshape(Bp, 1), table)


def _pick_bn(M):
    for b in (512, 256, 128, 64, 32, 16, 8):
        if M % b == 0:
            return b
    raise ValueError(M)


def _tc_matmul(parts, bias, nvalid=None):
    M = parts[0][0].shape[0]
    F = parts[0][1].shape[1]
    BN = _pick_bn(M)
    stats = nvalid is not None
    meta = tuple(p[2] is not None for p in parts)

    def body(*refs):
        i = pl.program_id(0)
        it = iter(refs)
        acc = None
        for has_pro in meta:
            x = next(it)[...]
            w = next(it)[...]
            if has_pro:
                sc, sh, sl = next(it)[...], next(it)[...], next(it)[...]
                v = x * sc + sh
                x = jnp.maximum(v, 0.0) + sl * jnp.minimum(v, 0.0)
            d = jnp.dot(x, w, preferred_element_type=jnp.float32)
            acc = d if acc is None else acc + d
        y = acc + next(it)[...]
        y_ref = next(it)
        y_ref[...] = y
        if stats:
            s_ref = next(it)
            ss_ref = next(it)
            rows = lax.broadcasted_iota(jnp.int32, (BN, 1), 0) + i * BN
            m = (rows < nvalid).astype(jnp.float32)
            ym = y * m
            ps = jnp.sum(ym, axis=0, keepdims=True)
            pss = jnp.sum(ym * ym, axis=0, keepdims=True)

            @pl.when(i == 0)
            def _():
                s_ref[...] = jnp.zeros_like(s_ref)
                ss_ref[...] = jnp.zeros_like(ss_ref)

            s_ref[...] += ps
            ss_ref[...] += pss

    in_specs = []
    args = []
    for X, W, pro in parts:
        K = X.shape[1]
        in_specs.append(pl.BlockSpec((BN, K), lambda i: (i, 0)))
        in_specs.append(pl.BlockSpec((K, F), lambda i: (0, 0)))
        args += [X, W]
        if pro is not None:
            for p in pro:
                in_specs.append(pl.BlockSpec((1, K), lambda i: (0, 0)))
                args.append(p)
    in_specs.append(pl.BlockSpec((1, F), lambda i: (0, 0)))
    args.append(bias.reshape(1, F))

    out_shapes = [jax.ShapeDtypeStruct((M, F), jnp.float32)]
    out_specs = [pl.BlockSpec((BN, F), lambda i: (i, 0))]
    if stats:
        out_shapes += [jax.ShapeDtypeStruct((1, F), jnp.float32)] * 2
        out_specs += [pl.BlockSpec((1, F), lambda i: (0, 0))] * 2

    res = pl.pallas_call(
        body,
        grid=(M // BN,),
        in_specs=in_specs,
        out_specs=out_specs,
        out_shape=out_shapes,
    )(*args)
    if stats:
        return res[0], res[1], res[2]
    return res[0]


def _bn_finalize(s, ss, n, bnp):
    mu = s[0] / n
    var = jnp.maximum(ss[0] / n - mu * mu, 0.0)
    rstd = lax.rsqrt(var + _EPS)
    scale = bnp["g"] * rstd
    shift = bnp["b"] - mu * scale
    slope = jnp.full_like(scale, 0.2)
    return scale, shift, slope


def _pro2d(pro, reps=1):
    return tuple(jnp.tile(p, reps).reshape(1, -1) for p in pro)


def _pad_idx(a, Bp):
    B = a.shape[0]
    return jnp.pad(a, (0, Bp - B)) if Bp != B else a


def _drop_self(no, n):
    return no.reshape(n, 7)[:, :6].reshape(-1)


def _pool_matrix(f):
    P = np.zeros((7 * f, f), np.float32)
    P[np.arange(7 * f), np.arange(7 * f) // 7] = 1.0 / 7.0
    return jnp.asarray(P)


def _updown_matrix(f):
    Q = np.zeros((2 * f, f), np.float32)
    Q[np.arange(2 * f), np.arange(2 * f) // 2] = 0.5
    return jnp.asarray(Q)


_TC_GATHER_FLOPS = 3e10


def _tc_conv_fused(table, no6p, W, bias, npad, n, pro, stats):
    V, f = table.shape
    fout = W.shape[1]
    idx2 = no6p.reshape(npad, 6)
    BN = _pick_bn(npad)
    W6n = W[: 6 * f].reshape(6, f, fout).transpose(1, 0, 2).reshape(f, 6 * fout)
    Wself = W[6 * f :]
    has_pro = pro is not None

    def body(*refs):
        it = iter(refs)
        idx_ref = next(it)
        tfull_ref = next(it)
        tblk_ref = next(it)
        w6_ref = next(it)
        ws_ref = next(it)
        b_ref = next(it)
        if has_pro:
            sc_ref, sh_ref, sl_ref = next(it), next(it), next(it)
        y_ref = next(it)
        if stats:
            s_ref, ss_ref = next(it), next(it)
        T_ref = next(it)
        i = pl.program_id(0)

        def act(v):
            if not has_pro:
                return v
            u = v * sc_ref[...] + sh_ref[...]
            return jnp.maximum(u, 0.0) + sl_ref[...] * jnp.minimum(u, 0.0)

        @pl.when(i == 0)
        def _():
            T_ref[...] = jnp.dot(
                act(tfull_ref[...]), w6_ref[...],
                preferred_element_type=jnp.float32,
            )

        iota = lax.broadcasted_iota(jnp.int32, (BN, V), 1)
        acc = jnp.dot(
            act(tblk_ref[...]), ws_ref[...], preferred_element_type=jnp.float32
        )
        for k in range(6):
            M = (iota == idx_ref[:, k : k + 1]).astype(jnp.float32)
            acc = acc + jnp.dot(
                M,
                T_ref[:, k * fout : (k + 1) * fout],
                preferred_element_type=jnp.float32,
            )
        y = acc + b_ref[...]
        y_ref[...] = y
        if stats:
            rows = lax.broadcasted_iota(jnp.int32, (BN, 1), 0) + i * BN
            m = (rows < n).astype(jnp.float32)
            ym = y * m
            ps = jnp.sum(ym, axis=0, keepdims=True)
            pss = jnp.sum(ym * ym, axis=0, keepdims=True)

            @pl.when(i == 0)
            def _():
                s_ref[...] = jnp.zeros_like(s_ref)
                ss_ref[...] = jnp.zeros_like(ss_ref)

            s_ref[...] += ps
            ss_ref[...] += pss

    in_specs = [
        pl.BlockSpec((BN, 6), lambda i: (i, 0)),
        pl.BlockSpec((V, f), lambda i: (0, 0)),
        pl.BlockSpec((BN, f), lambda i: (i, 0)),
        pl.BlockSpec((f, 6 * fout), lambda i: (0, 0)),
        pl.BlockSpec((f, fout), lambda i: (0, 0)),
        pl.BlockSpec((1, fout), lambda i: (0, 0)),
    ]
    args = [idx2, table, table, W6n, Wself, bias.reshape(1, fout)]
    if has_pro:
        for p in _pro2d(pro):
            in_specs.append(pl.BlockSpec((1, f), lambda i: (0, 0)))
            args.append(p)
    out_shapes = [jax.ShapeDtypeStruct((npad, fout), jnp.float32)]
    out_specs = [pl.BlockSpec((BN, fout), lambda i: (i, 0))]
    if stats:
        out_shapes += [jax.ShapeDtypeStruct((1, fout), jnp.float32)] * 2
        out_specs += [pl.BlockSpec((1, fout), lambda i: (0, 0))] * 2

    res = pl.pallas_call(
        body,
        grid=(npad // BN,),
        in_specs=in_specs,
        out_specs=out_specs,
        out_shape=out_shapes,
        scratch_shapes=[pltpu.VMEM((V, 6 * fout), jnp.float32)],
    )(*args)
    if stats:
        return res[0], res[1], res[2]
    return res[0]


def _conv(table, no_idx, W, b, npad, n, pro, stats=True):
    no6_pad = no_idx[0] if isinstance(no_idx, tuple) else no_idx
    V, f = table.shape
    fout = W.shape[1]
    unfused = 12.0 * npad * V * f
    fused = 12.0 * npad * V * fout + 12.0 * V * f * fout
    pro6 = _pro2d(pro, 6) if pro is not None else None
    pro1 = _pro2d(pro) if pro is not None else None
    if min(unfused, fused) <= _TC_GATHER_FLOPS:
        if fused <= unfused * 1.25 + 2e9:
            return _tc_conv_fused(table, no6_pad, W, b, npad, n, pro, stats)
        parts = [
            (_tc_gather6(table, no6_pad, npad), W[: 6 * f], pro6),
            (table, W[6 * f :], pro1),
        ]
    elif isinstance(no_idx, tuple) and no6_pad.shape[0] >= 200000:
        _, no3a, no3b = no_idx
        pro3 = _pro2d(pro, 3) if pro is not None else None
        Ga = _sc_gather(table, no3a).reshape(npad, 3 * f)
        Gb = _sc_gather(table, no3b).reshape(npad, 3 * f)
        parts = [
            (Ga, W[: 3 * f], pro3),
            (Gb, W[3 * f : 6 * f], pro3),
            (table, W[6 * f :], pro1),
        ]
    else:
        parts = [
            (_sc_gather(table, no6_pad).reshape(npad, 6 * f), W[: 6 * f], pro6),
            (table, W[6 * f :], pro1),
        ]
    return _tc_matmul(parts, b, nvalid=n) if stats else _tc_matmul(parts, b)


def _double_conv(table, n, npad, no6_pad, p, pro_in):
    y1, s1, ss1 = _conv(table, no6_pad, p["c1"]["W"], p["c1"]["b"], npad, n, pro_in)
    pro1 = _bn_finalize(s1, ss1, n, p["bn1"])
    y2, s2, ss2 = _conv(y1, no6_pad, p["c2"]["W"], p["c2"]["b"], npad, n, pro1)
    pro2 = _bn_finalize(s2, ss2, n, p["bn2"])
    return y2, pro2


def kernel(x, params, idx):
    levels = _LEVELS
    npads = [_rup(n, 512) for n in levels]
    no6_pad = []
    for i, n in enumerate(levels):
        no2d = idx["neigh_%d" % n].reshape(n, 7)[:, :6]
        no6p = _pad_idx(no2d.reshape(-1), npads[i] * 6)
        if n * 6 >= 200000:
            no3a = _pad_idx(no2d[:, :3].reshape(-1), npads[i] * 3)
            no3b = _pad_idx(no2d[:, 3:].reshape(-1), npads[i] * 3)
            no6_pad.append((no6p, no3a, no3b))
        else:
            no6_pad.append(no6p)

    x16 = jnp.pad(x, ((0, npads[0] - levels[0]), (0, 13)))
    W1 = params["down1"]["c1"]["W"].reshape(7, 3, -1)
    W1p = jnp.zeros((7, 16, W1.shape[2]), jnp.float32).at[:, :3, :].set(W1)
    W1p = W1p.reshape(7 * 16, -1)
    p1 = {
        "c1": {"W": W1p, "b": params["down1"]["c1"]["b"]},
        "bn1": params["down1"]["bn1"],
        "c2": params["down1"]["c2"],
        "bn2": params["down1"]["bn2"],
    }

    skips = []
    table, pro_in = x16, None
    for i in range(5):
        n, npad = levels[i], npads[i]
        p = p1 if i == 0 else params["down%d" % (i + 1)]
        y, pro = _double_conv(table, n, npad, no6_pad[i], p, pro_in)
        if i < 4:
            skips.append((y, pro))
            r, rpad = levels[i + 1], npads[i + 1]
            f = y.shape[1]
            pno6 = _pad_idx(
                _drop_self(idx["neigh_%d" % n][: r * 7], r), rpad * 6
            )
            table = _conv(
                y, pno6, _pool_matrix(f), jnp.zeros((f,), jnp.float32),
                rpad, 0, pro, stats=False,
            )
            pro_in = None
        else:
            h, pro_h = y, pro

    for j in range(1, 5):
        lev = 4 - j
        n, npad = levels[lev], npads[lev]
        r = (n + 6) // 4
        p = params["up%d" % j]
        fout = p["c1"]["W"].shape[1]

        y_up = _tc_matmul(
            [(h, p["up"]["W"], _pro2d(pro_h))], p["up"]["b"]
        )
        y_rows = y_up.reshape(-1, fout)

        top = idx["up_top_%d" % n]
        down = idx["up_down_%d" % n]
        Bp_t = _rup(top.shape[0], 8 * _NW)
        Bp_d = _rup(down.shape[0], 8 * _NW)
        idx_td = jnp.concatenate([_pad_idx(top, Bp_t), _pad_idx(down, Bp_d)])
        Vr = y_rows.shape[0]
        if 2.0 * idx_td.shape[0] * Vr * fout <= _TC_GATHER_FLOPS:
            G_td = _tc_gather1(y_rows, idx_td)
        else:
            G_td = _sc_gather(y_rows, idx_td)
        x1 = G_td[:r]
        Gd = G_td[Bp_t:].reshape(Bp_d // 2, 2 * fout)
        x2 = _tc_matmul(
            [(Gd, _updown_matrix(fout), None)], jnp.zeros((fout,), jnp.float32)
        )
        h_up = jnp.concatenate([x1, x2[: n - r]], axis=0)

        y_skip, pro_skip = skips[lev]
        hc = jnp.concatenate([h_up, y_skip[:n]], axis=1)
        hc = jnp.pad(hc, ((0, npad - n), (0, 0)))
        ones = jnp.ones((fout,), jnp.float32)
        zeros = jnp.zeros((fout,), jnp.float32)
        pro_c = (
            jnp.concatenate([ones, pro_skip[0]]),
            jnp.concatenate([zeros, pro_skip[1]]),
            jnp.concatenate([ones, pro_skip[2]]),
        )
        h, pro_h = _double_conv(hc, n, npad, no6_pad[lev], p, pro_c)

    out = _tc_matmul(
        [(h, params["outc"]["W"], _pro2d(pro_h))], params["outc"]["b"]
    )
    return out[: levels[0]]


# speedup vs baseline: 1.0042x; 1.0042x over previous
import functools

import numpy as np

import jax
import jax.numpy as jnp
from jax import lax
from jax.experimental import pallas as pl
from jax.experimental.pallas import tpu as pltpu
from jax.experimental.pallas import tpu_sc as plsc

_LEVELS = [40962, 10242, 2562, 642, 162]
_EPS = 1e-5

_NC = 2
_NS = 16
_NW = _NC * _NS


def _rup(x, m):
    return (x + m - 1) // m * m


@functools.lru_cache(maxsize=None)
def _make_sc_gather(V, D, Bp):
    assert Bp % (8 * _NW) == 0
    b_per_w = Bp // _NW
    C = min(128 if D <= 256 else 64, b_per_w)
    NBUF = max(1, min(8, 393216 // (C * D * 4)))
    nfull = b_per_w // C
    tail = b_per_w % C
    ngrp = nfull // NBUF
    nrem = nfull % NBUF

    mesh = plsc.VectorSubcoreMesh(core_axis_name="c", subcore_axis_name="s")
    scratch = [pltpu.VMEM((b_per_w,), jnp.int32)]
    scratch += [pltpu.VMEM((C, D), jnp.float32) for _ in range(NBUF)]
    scratch += [pltpu.SemaphoreType.DMA for _ in range(2 * NBUF)]

    @functools.partial(
        pl.kernel,
        mesh=mesh,
        out_type=jax.ShapeDtypeStruct((Bp, D), jnp.float32),
        compiler_params=pltpu.CompilerParams(use_tc_tiling_on_sc=False),
        scratch_types=scratch,
    )
    def gather_kernel(table_hbm, idx_hbm, out_hbm, idx_v, *rest):
        bufs = rest[:NBUF]
        gsem = rest[NBUF : 2 * NBUF]
        wsem = rest[2 * NBUF : 3 * NBUF]
        wid = lax.axis_index("s") * _NC + lax.axis_index("c")
        base = wid * b_per_w
        pltpu.sync_copy(idx_hbm.at[pl.ds(base, b_per_w)], idx_v)

        def fire_gather(off, b):
            pltpu.async_copy(
                table_hbm.at[idx_v.at[pl.ds(off, C)]], bufs[b], gsem[b]
            )

        def wait_gather(b):
            pltpu.make_async_copy(
                table_hbm.at[idx_v.at[pl.ds(0, C)]], bufs[b], gsem[b]
            ).wait()

        def fire_wb(off, b):
            pltpu.async_copy(bufs[b], out_hbm.at[pl.ds(base + off, C)], wsem[b])

        def wait_wb(b):
            pltpu.make_async_copy(bufs[b], out_hbm.at[pl.ds(0, C)], wsem[b]).wait()

        def group(g, carry):
            for b in range(NBUF):
                c = g * NBUF + b

                @pl.when(g > 0)
                def _(b=b):
                    wait_wb(b)

                fire_gather(c * C, b)
                if b > 0:
                    wait_gather(b - 1)
                    fire_wb((c - 1) * C, b - 1)
                else:

                    @pl.when(g > 0)
                    def _(c=c):
                        wait_gather(NBUF - 1)
                        fire_wb((c - 1) * C, NBUF - 1)

            return carry

        if ngrp > 0:
            lax.fori_loop(0, ngrp, group, 0)
            wait_gather(NBUF - 1)
            fire_wb((ngrp * NBUF - 1) * C, NBUF - 1)
            for b in range(NBUF):
                wait_wb(b)

        off0 = ngrp * NBUF * C
        for j in range(nrem):
            off = off0 + j * C
            pltpu.async_copy(
                table_hbm.at[idx_v.at[pl.ds(off, C)]], bufs[0], gsem[0]
            ).wait()
            pltpu.sync_copy(bufs[0], out_hbm.at[pl.ds(base + off, C)])
        if tail:
            off = nfull * C
            pltpu.async_copy(
                table_hbm.at[idx_v.at[pl.ds(off, tail)]],
                bufs[0].at[pl.ds(0, tail)],
                gsem[0],
            ).wait()
            pltpu.sync_copy(
                bufs[0].at[pl.ds(0, tail)], out_hbm.at[pl.ds(base + off, tail)]
            )

    return gather_kernel


def _sc_gather(table, idxp):
    V, D = table.shape
    (Bp,) = idxp.shape
    return _make_sc_gather(V, D, Bp)(table, idxp)


def _tc_gather6(table, no6p, npad):
    V, f = table.shape
    idx2 = no6p.reshape(npad, 6)
    BN = _pick_bn(npad)

    def body(idx_ref, t_ref, out_ref):
        iota = lax.broadcasted_iota(jnp.int32, (BN, V), 1)
        t = t_ref[...]
        for k in range(6):
            sel = idx_ref[:, k : k + 1]
            M = (iota == sel).astype(jnp.float32)
            out_ref[:, k * f : (k + 1) * f] = jnp.dot(
                M, t, preferred_element_type=jnp.float32
            )

    return pl.pallas_call(
        body,
        grid=(npad // BN,),
        in_specs=[
            pl.BlockSpec((BN, 6), lambda i: (i, 0)),
            pl.BlockSpec((V, f), lambda i: (0, 0)),
        ],
        out_specs=pl.BlockSpec((BN, 6 * f), lambda i: (i, 0)),
        out_shape=jax.ShapeDtypeStruct((npad, 6 * f), jnp.float32),
    )(idx2, table)


def _tc_gather1(table, idxp):
    V, f = table.shape
    (Bp,) = idxp.shape
    BN = _pick_bn(Bp)

    def body(idx_ref, t_ref, out_ref):
        iota = lax.broadcasted_iota(jnp.int32, (BN, V), 1)
        M = (iota == idx_ref[...]).astype(jnp.float32)
        out_ref[...] = jnp.dot(M, t_ref[...], preferred_element_type=jnp.float32)

    return pl.pallas_call(
        body,
        grid=(Bp // BN,),
        in_specs=[
            pl.BlockSpec((BN, 1), lambda i: (i, 0)),
            pl.BlockSpec((V, f), lambda i: (0, 0)),
        ],
        out_specs=pl.BlockSpec((BN, f), lambda i: (i, 0)),
        out_shape=jax.ShapeDtypeStruct((Bp, f), jnp.float32),
    )(idxp.reshape(Bp, 1), table)


def _pick_bn(M):
    for b in (512, 256, 128, 64, 32, 16, 8):
        if M % b == 0:
            return b
    raise ValueError(M)


def _tc_matmul(parts, bias, nvalid=None):
    M = parts[0][0].shape[0]
    F = parts[0][1].shape[1]
    BN = _pick_bn(M)
    stats = nvalid is not None
    meta = tuple(p[2] is not None for p in parts)

    def body(*refs):
        i = pl.program_id(0)
        it = iter(refs)
        acc = None
        for has_pro in meta:
            x = next(it)[...]
            w = next(it)[...]
            if has_pro:
                sc, sh, sl = next(it)[...], next(it)[...], next(it)[...]
                v = x * sc + sh
                x = jnp.maximum(v, 0.0) + sl * jnp.minimum(v, 0.0)
            d = jnp.dot(x, w, preferred_element_type=jnp.float32)
            acc = d if acc is None else acc + d
        y = acc + next(it)[...]
        y_ref = next(it)
        y_ref[...] = y
        if stats:
            s_ref = next(it)
            ss_ref = next(it)
            rows = lax.broadcasted_iota(jnp.int32, (BN, 1), 0) + i * BN
            m = (rows < nvalid).astype(jnp.float32)
            ym = y * m
            ps = jnp.sum(ym, axis=0, keepdims=True)
            pss = jnp.sum(ym * ym, axis=0, keepdims=True)

            @pl.when(i == 0)
            def _():
                s_ref[...] = jnp.zeros_like(s_ref)
                ss_ref[...] = jnp.zeros_like(ss_ref)

            s_ref[...] += ps
            ss_ref[...] += pss

    in_specs = []
    args = []
    for X, W, pro in parts:
        K = X.shape[1]
        in_specs.append(pl.BlockSpec((BN, K), lambda i: (i, 0)))
        in_specs.append(pl.BlockSpec((K, F), lambda i: (0, 0)))
        args += [X, W]
        if pro is not None:
            for p in pro:
                in_specs.append(pl.BlockSpec((1, K), lambda i: (0, 0)))
                args.append(p)
    in_specs.append(pl.BlockSpec((1, F), lambda i: (0, 0)))
    args.append(bias.reshape(1, F))

    out_shapes = [jax.ShapeDtypeStruct((M, F), jnp.float32)]
    out_specs = [pl.BlockSpec((BN, F), lambda i: (i, 0))]
    if stats:
        out_shapes += [jax.ShapeDtypeStruct((1, F), jnp.float32)] * 2
        out_specs += [pl.BlockSpec((1, F), lambda i: (0, 0))] * 2

    res = pl.pallas_call(
        body,
        grid=(M // BN,),
        in_specs=in_specs,
        out_specs=out_specs,
        out_shape=out_shapes,
    )(*args)
    if stats:
        return res[0], res[1], res[2]
    return res[0]


def _bn_finalize(s, ss, n, bnp):
    mu = s[0] / n
    var = jnp.maximum(ss[0] / n - mu * mu, 0.0)
    rstd = lax.rsqrt(var + _EPS)
    scale = bnp["g"] * rstd
    shift = bnp["b"] - mu * scale
    slope = jnp.full_like(scale, 0.2)
    return scale, shift, slope


def _pro2d(pro, reps=1):
    return tuple(jnp.tile(p, reps).reshape(1, -1) for p in pro)


def _pad_idx(a, Bp):
    B = a.shape[0]
    return jnp.pad(a, (0, Bp - B)) if Bp != B else a


def _drop_self(no, n):
    return no.reshape(n, 7)[:, :6].reshape(-1)


def _pool_matrix(f):
    P = np.zeros((7 * f, f), np.float32)
    P[np.arange(7 * f), np.arange(7 * f) // 7] = 1.0 / 7.0
    return jnp.asarray(P)


def _updown_matrix(f):
    Q = np.zeros((2 * f, f), np.float32)
    Q[np.arange(2 * f), np.arange(2 * f) // 2] = 0.5
    return jnp.asarray(Q)


_TC_GATHER_FLOPS = 3e10


def _tc_conv_fused(table, no6p, W, bias, npad, n, pro, stats):
    V, f = table.shape
    fout = W.shape[1]
    idx2 = no6p.reshape(npad, 6)
    BN = _pick_bn(npad)
    W6n = W[: 6 * f].reshape(6, f, fout).transpose(1, 0, 2).reshape(f, 6 * fout)
    Wself = W[6 * f :]
    has_pro = pro is not None

    def body(*refs):
        it = iter(refs)
        idx_ref = next(it)
        tfull_ref = next(it)
        tblk_ref = next(it)
        w6_ref = next(it)
        ws_ref = next(it)
        b_ref = next(it)
        if has_pro:
            sc_ref, sh_ref, sl_ref = next(it), next(it), next(it)
        y_ref = next(it)
        if stats:
            s_ref, ss_ref = next(it), next(it)
        T_ref = next(it)
        i = pl.program_id(0)

        def act(v):
            if not has_pro:
                return v
            u = v * sc_ref[...] + sh_ref[...]
            return jnp.maximum(u, 0.0) + sl_ref[...] * jnp.minimum(u, 0.0)

        @pl.when(i == 0)
        def _():
            T_ref[...] = jnp.dot(
                act(tfull_ref[...]), w6_ref[...],
                preferred_element_type=jnp.float32,
            )

        iota = lax.broadcasted_iota(jnp.int32, (BN, V), 1)
        acc = jnp.dot(
            act(tblk_ref[...]), ws_ref[...], preferred_element_type=jnp.float32
        )
        for k in range(6):
            M = (iota == idx_ref[:, k : k + 1]).astype(jnp.float32)
            acc = acc + jnp.dot(
                M,
                T_ref[:, k * fout : (k + 1) * fout],
                preferred_element_type=jnp.float32,
            )
        y = acc + b_ref[...]
        y_ref[...] = y
        if stats:
            rows = lax.broadcasted_iota(jnp.int32, (BN, 1), 0) + i * BN
            m = (rows < n).astype(jnp.float32)
            ym = y * m
            ps = jnp.sum(ym, axis=0, keepdims=True)
            pss = jnp.sum(ym * ym, axis=0, keepdims=True)

            @pl.when(i == 0)
            def _():
                s_ref[...] = jnp.zeros_like(s_ref)
                ss_ref[...] = jnp.zeros_like(ss_ref)

            s_ref[...] += ps
            ss_ref[...] += pss

    in_specs = [
        pl.BlockSpec((BN, 6), lambda i: (i, 0)),
        pl.BlockSpec((V, f), lambda i: (0, 0)),
        pl.BlockSpec((BN, f), lambda i: (i, 0)),
        pl.BlockSpec((f, 6 * fout), lambda i: (0, 0)),
        pl.BlockSpec((f, fout), lambda i: (0, 0)),
        pl.BlockSpec((1, fout), lambda i: (0, 0)),
    ]
    args = [idx2, table, table, W6n, Wself, bias.reshape(1, fout)]
    if has_pro:
        for p in _pro2d(pro):
            in_specs.append(pl.BlockSpec((1, f), lambda i: (0, 0)))
            args.append(p)
    out_shapes = [jax.ShapeDtypeStruct((npad, fout), jnp.float32)]
    out_specs = [pl.BlockSpec((BN, fout), lambda i: (i, 0))]
    if stats:
        out_shapes += [jax.ShapeDtypeStruct((1, fout), jnp.float32)] * 2
        out_specs += [pl.BlockSpec((1, fout), lambda i: (0, 0))] * 2

    res = pl.pallas_call(
        body,
        grid=(npad // BN,),
        in_specs=in_specs,
        out_specs=out_specs,
        out_shape=out_shapes,
        scratch_shapes=[pltpu.VMEM((V, 6 * fout), jnp.float32)],
    )(*args)
    if stats:
        return res[0], res[1], res[2]
    return res[0]


def _conv(table, no_idx, W, b, npad, n, pro, stats=True):
    no6_pad = no_idx[0] if isinstance(no_idx, tuple) else no_idx
    V, f = table.shape
    fout = W.shape[1]
    unfused = 12.0 * npad * V * f
    fused = 12.0 * npad * V * fout + 12.0 * V * f * fout
    pro6 = _pro2d(pro, 6) if pro is not None else None
    pro1 = _pro2d(pro) if pro is not None else None
    if min(unfused, fused) <= _TC_GATHER_FLOPS:
        if fused <= unfused * 1.25 + 2e9:
            return _tc_conv_fused(table, no6_pad, W, b, npad, n, pro, stats)
        parts = [
            (_tc_gather6(table, no6_pad, npad), W[: 6 * f], pro6),
            (table, W[6 * f :], pro1),
        ]
    else:
        parts = [
            (_sc_gather(table, no6_pad).reshape(npad, 6 * f), W[: 6 * f], pro6),
            (table, W[6 * f :], pro1),
        ]
    return _tc_matmul(parts, b, nvalid=n) if stats else _tc_matmul(parts, b)


def _double_conv(table, n, npad, no6_pad, p, pro_in):
    y1, s1, ss1 = _conv(table, no6_pad, p["c1"]["W"], p["c1"]["b"], npad, n, pro_in)
    pro1 = _bn_finalize(s1, ss1, n, p["bn1"])
    y2, s2, ss2 = _conv(y1, no6_pad, p["c2"]["W"], p["c2"]["b"], npad, n, pro1)
    pro2 = _bn_finalize(s2, ss2, n, p["bn2"])
    return y2, pro2


def kernel(x, params, idx):
    levels = _LEVELS
    npads = [_rup(n, 512) for n in levels]
    no6_pad = []
    for i, n in enumerate(levels):
        no2d = idx["neigh_%d" % n].reshape(n, 7)[:, :6]
        no6_pad.append(_pad_idx(no2d.reshape(-1), npads[i] * 6))

    x16 = jnp.pad(x, ((0, npads[0] - levels[0]), (0, 13)))
    W1 = params["down1"]["c1"]["W"].reshape(7, 3, -1)
    W1p = jnp.zeros((7, 16, W1.shape[2]), jnp.float32).at[:, :3, :].set(W1)
    W1p = W1p.reshape(7 * 16, -1)
    p1 = {
        "c1": {"W": W1p, "b": params["down1"]["c1"]["b"]},
        "bn1": params["down1"]["bn1"],
        "c2": params["down1"]["c2"],
        "bn2": params["down1"]["bn2"],
    }

    skips = []
    table, pro_in = x16, None
    for i in range(5):
        n, npad = levels[i], npads[i]
        p = p1 if i == 0 else params["down%d" % (i + 1)]
        y, pro = _double_conv(table, n, npad, no6_pad[i], p, pro_in)
        if i < 4:
            skips.append((y, pro))
            r, rpad = levels[i + 1], npads[i + 1]
            f = y.shape[1]
            pno6 = _pad_idx(
                _drop_self(idx["neigh_%d" % n][: r * 7], r), rpad * 6
            )
            table = _conv(
                y, pno6, _pool_matrix(f), jnp.zeros((f,), jnp.float32),
                rpad, 0, pro, stats=False,
            )
            pro_in = None
        else:
            h, pro_h = y, pro

    for j in range(1, 5):
        lev = 4 - j
        n, npad = levels[lev], npads[lev]
        r = (n + 6) // 4
        p = params["up%d" % j]
        fout = p["c1"]["W"].shape[1]

        y_up = _tc_matmul(
            [(h, p["up"]["W"], _pro2d(pro_h))], p["up"]["b"]
        )
        y_rows = y_up.reshape(-1, fout)

        top = idx["up_top_%d" % n]
        down = idx["up_down_%d" % n]
        Bp_t = _rup(top.shape[0], 8 * _NW)
        Bp_d = _rup(down.shape[0], 8 * _NW)
        idx_td = jnp.concatenate([_pad_idx(top, Bp_t), _pad_idx(down, Bp_d)])
        Vr = y_rows.shape[0]
        if 2.0 * idx_td.shape[0] * Vr * fout <= _TC_GATHER_FLOPS:
            G_td = _tc_gather1(y_rows, idx_td)
        else:
            G_td = _sc_gather(y_rows, idx_td)
        x1 = G_td[:r]
        Gd = G_td[Bp_t:].reshape(Bp_d // 2, 2 * fout)
        x2 = _tc_matmul(
            [(Gd, _updown_matrix(fout), None)], jnp.zeros((fout,), jnp.float32)
        )
        h_up = jnp.concatenate(
            [x1, x2[: n - r], jnp.zeros((npad - n, fout), jnp.float32)], axis=0
        )

        y_skip, pro_skip = skips[lev]
        hc = jnp.concatenate([h_up, y_skip], axis=1)
        ones = jnp.ones((fout,), jnp.float32)
        zeros = jnp.zeros((fout,), jnp.float32)
        pro_c = (
            jnp.concatenate([ones, pro_skip[0]]),
            jnp.concatenate([zeros, pro_skip[1]]),
            jnp.concatenate([ones, pro_skip[2]]),
        )
        h, pro_h = _double_conv(hc, n, npad, no6_pad[lev], p, pro_c)

    out = _tc_matmul(
        [(h, params["outc"]["W"], _pro2d(pro_h))], params["outc"]["b"]
    )
    return out[: levels[0]]


# speedup vs baseline: 1.0208x; 1.0165x over previous
import functools

import numpy as np

import jax
import jax.numpy as jnp
from jax import lax
from jax.experimental import pallas as pl
from jax.experimental.pallas import tpu as pltpu
from jax.experimental.pallas import tpu_sc as plsc

_LEVELS = [40962, 10242, 2562, 642, 162]
_EPS = 1e-5

_NC = 2
_NS = 16
_NW = _NC * _NS


def _rup(x, m):
    return (x + m - 1) // m * m


@functools.lru_cache(maxsize=None)
def _make_sc_gather(V, D, Bp):
    assert Bp % (8 * _NW) == 0
    b_per_w = Bp // _NW
    C = min(128 if D <= 256 else 64, b_per_w)
    NBUF = max(1, min(8, 393216 // (C * D * 4)))
    nfull = b_per_w // C
    tail = b_per_w % C
    ngrp = nfull // NBUF
    nrem = nfull % NBUF

    mesh = plsc.VectorSubcoreMesh(core_axis_name="c", subcore_axis_name="s")
    scratch = [pltpu.VMEM((b_per_w,), jnp.int32)]
    scratch += [pltpu.VMEM((C, D), jnp.float32) for _ in range(NBUF)]
    scratch += [pltpu.SemaphoreType.DMA for _ in range(2 * NBUF)]

    @functools.partial(
        pl.kernel,
        mesh=mesh,
        out_type=jax.ShapeDtypeStruct((Bp, D), jnp.float32),
        compiler_params=pltpu.CompilerParams(use_tc_tiling_on_sc=False),
        scratch_types=scratch,
    )
    def gather_kernel(table_hbm, idx_hbm, out_hbm, idx_v, *rest):
        bufs = rest[:NBUF]
        gsem = rest[NBUF : 2 * NBUF]
        wsem = rest[2 * NBUF : 3 * NBUF]
        wid = lax.axis_index("s") * _NC + lax.axis_index("c")
        base = wid * b_per_w
        pltpu.sync_copy(idx_hbm.at[pl.ds(base, b_per_w)], idx_v)

        def fire_gather(off, b):
            pltpu.async_copy(
                table_hbm.at[idx_v.at[pl.ds(off, C)]], bufs[b], gsem[b]
            )

        def wait_gather(b):
            pltpu.make_async_copy(
                table_hbm.at[idx_v.at[pl.ds(0, C)]], bufs[b], gsem[b]
            ).wait()

        def fire_wb(off, b):
            pltpu.async_copy(bufs[b], out_hbm.at[pl.ds(base + off, C)], wsem[b])

        def wait_wb(b):
            pltpu.make_async_copy(bufs[b], out_hbm.at[pl.ds(0, C)], wsem[b]).wait()

        def group(g, carry):
            for b in range(NBUF):
                c = g * NBUF + b

                @pl.when(g > 0)
                def _(b=b):
                    wait_wb(b)

                fire_gather(c * C, b)
                if b > 0:
                    wait_gather(b - 1)
                    fire_wb((c - 1) * C, b - 1)
                else:

                    @pl.when(g > 0)
                    def _(c=c):
                        wait_gather(NBUF - 1)
                        fire_wb((c - 1) * C, NBUF - 1)

            return carry

        if ngrp > 0:
            lax.fori_loop(0, ngrp, group, 0)
            wait_gather(NBUF - 1)
            fire_wb((ngrp * NBUF - 1) * C, NBUF - 1)
            for b in range(NBUF):
                wait_wb(b)

        off0 = ngrp * NBUF * C
        for j in range(nrem):
            off = off0 + j * C
            pltpu.async_copy(
                table_hbm.at[idx_v.at[pl.ds(off, C)]], bufs[0], gsem[0]
            ).wait()
            pltpu.sync_copy(bufs[0], out_hbm.at[pl.ds(base + off, C)])
        if tail:
            off = nfull * C
            pltpu.async_copy(
                table_hbm.at[idx_v.at[pl.ds(off, tail)]],
                bufs[0].at[pl.ds(0, tail)],
                gsem[0],
            ).wait()
            pltpu.sync_copy(
                bufs[0].at[pl.ds(0, tail)], out_hbm.at[pl.ds(base + off, tail)]
            )

    return gather_kernel


def _sc_gather(table, idxp):
    V, D = table.shape
    (Bp,) = idxp.shape
    return _make_sc_gather(V, D, Bp)(table, idxp)


def _tc_gather6(table, no6p, npad):
    V, f = table.shape
    idx2 = no6p.reshape(npad, 6)
    BN = _pick_bn(npad)

    def body(idx_ref, t_ref, out_ref):
        iota = lax.broadcasted_iota(jnp.int32, (BN, V), 1)
        t = t_ref[...]
        for k in range(6):
            sel = idx_ref[:, k : k + 1]
            M = (iota == sel).astype(jnp.float32)
            out_ref[:, k * f : (k + 1) * f] = jnp.dot(
                M, t, preferred_element_type=jnp.float32
            )

    return pl.pallas_call(
        body,
        grid=(npad // BN,),
        in_specs=[
            pl.BlockSpec((BN, 6), lambda i: (i, 0)),
            pl.BlockSpec((V, f), lambda i: (0, 0)),
        ],
        out_specs=pl.BlockSpec((BN, 6 * f), lambda i: (i, 0)),
        out_shape=jax.ShapeDtypeStruct((npad, 6 * f), jnp.float32),
    )(idx2, table)


def _tc_gather1(table, idxp):
    V, f = table.shape
    (Bp,) = idxp.shape
    BN = _pick_bn(Bp)

    def body(idx_ref, t_ref, out_ref):
        iota = lax.broadcasted_iota(jnp.int32, (BN, V), 1)
        M = (iota == idx_ref[...]).astype(jnp.float32)
        out_ref[...] = jnp.dot(M, t_ref[...], preferred_element_type=jnp.float32)

    return pl.pallas_call(
        body,
        grid=(Bp // BN,),
        in_specs=[
            pl.BlockSpec((BN, 1), lambda i: (i, 0)),
            pl.BlockSpec((V, f), lambda i: (0, 0)),
        ],
        out_specs=pl.BlockSpec((BN, f), lambda i: (i, 0)),
        out_shape=jax.ShapeDtypeStruct((Bp, f), jnp.float32),
    )(idxp.reshape(Bp, 1), table)


def _pick_bn(M):
    for b in (512, 256, 128, 64, 32, 16, 8):
        if M % b == 0:
            return b
    raise ValueError(M)


def _tc_matmul(parts, bias, nvalid=None):
    M = parts[0][0].shape[0]
    F = parts[0][1].shape[1]
    BN = _pick_bn(M)
    stats = nvalid is not None
    meta = tuple(p[2] is not None for p in parts)

    def body(*refs):
        i = pl.program_id(0)
        it = iter(refs)
        acc = None
        for has_pro in meta:
            x = next(it)[...]
            w = next(it)[...]
            if has_pro:
                sc, sh, sl = next(it)[...], next(it)[...], next(it)[...]
                v = x * sc + sh
                x = jnp.maximum(v, 0.0) + sl * jnp.minimum(v, 0.0)
            d = jnp.dot(x, w, preferred_element_type=jnp.float32)
            acc = d if acc is None else acc + d
        y = acc + next(it)[...]
        y_ref = next(it)
        y_ref[...] = y
        if stats:
            s_ref = next(it)
            ss_ref = next(it)
            rows = lax.broadcasted_iota(jnp.int32, (BN, 1), 0) + i * BN
            m = (rows < nvalid).astype(jnp.float32)
            ym = y * m
            ps = jnp.sum(ym, axis=0, keepdims=True)
            pss = jnp.sum(ym * ym, axis=0, keepdims=True)

            @pl.when(i == 0)
            def _():
                s_ref[...] = jnp.zeros_like(s_ref)
                ss_ref[...] = jnp.zeros_like(ss_ref)

            s_ref[...] += ps
            ss_ref[...] += pss

    in_specs = []
    args = []
    for X, W, pro in parts:
        K = X.shape[1]
        in_specs.append(pl.BlockSpec((BN, K), lambda i: (i, 0)))
        in_specs.append(pl.BlockSpec((K, F), lambda i: (0, 0)))
        args += [X, W]
        if pro is not None:
            for p in pro:
                in_specs.append(pl.BlockSpec((1, K), lambda i: (0, 0)))
                args.append(p)
    in_specs.append(pl.BlockSpec((1, F), lambda i: (0, 0)))
    args.append(bias.reshape(1, F))

    out_shapes = [jax.ShapeDtypeStruct((M, F), jnp.float32)]
    out_specs = [pl.BlockSpec((BN, F), lambda i: (i, 0))]
    if stats:
        out_shapes += [jax.ShapeDtypeStruct((1, F), jnp.float32)] * 2
        out_specs += [pl.BlockSpec((1, F), lambda i: (0, 0))] * 2

    res = pl.pallas_call(
        body,
        grid=(M // BN,),
        in_specs=in_specs,
        out_specs=out_specs,
        out_shape=out_shapes,
    )(*args)
    if stats:
        return res[0], res[1], res[2]
    return res[0]


def _bn_finalize(s, ss, n, bnp):
    mu = s[0] / n
    var = jnp.maximum(ss[0] / n - mu * mu, 0.0)
    rstd = lax.rsqrt(var + _EPS)
    scale = bnp["g"] * rstd
    shift = bnp["b"] - mu * scale
    slope = jnp.full_like(scale, 0.2)
    return scale, shift, slope


def _pro2d(pro, reps=1):
    return tuple(jnp.tile(p, reps).reshape(1, -1) for p in pro)


def _pad_idx(a, Bp):
    B = a.shape[0]
    return jnp.pad(a, (0, Bp - B)) if Bp != B else a


def _drop_self(no, n):
    return no.reshape(n, 7)[:, :6].reshape(-1)


def _pool_matrix(f):
    P = np.zeros((7 * f, f), np.float32)
    P[np.arange(7 * f), np.arange(7 * f) // 7] = 1.0 / 7.0
    return jnp.asarray(P)


def _updown_matrix(f):
    Q = np.zeros((2 * f, f), np.float32)
    Q[np.arange(2 * f), np.arange(2 * f) // 2] = 0.5
    return jnp.asarray(Q)


_TC_GATHER_FLOPS = 3e10


def _tc_conv_fused(table, no6p, W, bias, npad, n, pro, stats):
    V, f = table.shape
    fout = W.shape[1]
    idx2 = no6p.reshape(npad, 6)
    BN = _pick_bn(npad)
    W6n = W[: 6 * f].reshape(6, f, fout).transpose(1, 0, 2).reshape(f, 6 * fout)
    Wself = W[6 * f :]
    has_pro = pro is not None

    def body(*refs):
        it = iter(refs)
        idx_ref = next(it)
        tfull_ref = next(it)
        tblk_ref = next(it)
        w6_ref = next(it)
        ws_ref = next(it)
        b_ref = next(it)
        if has_pro:
            sc_ref, sh_ref, sl_ref = next(it), next(it), next(it)
        y_ref = next(it)
        if stats:
            s_ref, ss_ref = next(it), next(it)
        T_ref = next(it)
        i = pl.program_id(0)

        def act(v):
            if not has_pro:
                return v
            u = v * sc_ref[...] + sh_ref[...]
            return jnp.maximum(u, 0.0) + sl_ref[...] * jnp.minimum(u, 0.0)

        @pl.when(i == 0)
        def _():
            T_ref[...] = jnp.dot(
                act(tfull_ref[...]), w6_ref[...],
                preferred_element_type=jnp.float32,
            )

        iota = lax.broadcasted_iota(jnp.int32, (BN, V), 1)
        acc = jnp.dot(
            act(tblk_ref[...]), ws_ref[...], preferred_element_type=jnp.float32
        )
        for k in range(6):
            M = (iota == idx_ref[:, k : k + 1]).astype(jnp.float32)
            acc = acc + jnp.dot(
                M,
                T_ref[:, k * fout : (k + 1) * fout],
                preferred_element_type=jnp.float32,
            )
        y = acc + b_ref[...]
        y_ref[...] = y
        if stats:
            rows = lax.broadcasted_iota(jnp.int32, (BN, 1), 0) + i * BN
            m = (rows < n).astype(jnp.float32)
            ym = y * m
            ps = jnp.sum(ym, axis=0, keepdims=True)
            pss = jnp.sum(ym * ym, axis=0, keepdims=True)

            @pl.when(i == 0)
            def _():
                s_ref[...] = jnp.zeros_like(s_ref)
                ss_ref[...] = jnp.zeros_like(ss_ref)

            s_ref[...] += ps
            ss_ref[...] += pss

    in_specs = [
        pl.BlockSpec((BN, 6), lambda i: (i, 0)),
        pl.BlockSpec((V, f), lambda i: (0, 0)),
        pl.BlockSpec((BN, f), lambda i: (i, 0)),
        pl.BlockSpec((f, 6 * fout), lambda i: (0, 0)),
        pl.BlockSpec((f, fout), lambda i: (0, 0)),
        pl.BlockSpec((1, fout), lambda i: (0, 0)),
    ]
    args = [idx2, table, table, W6n, Wself, bias.reshape(1, fout)]
    if has_pro:
        for p in _pro2d(pro):
            in_specs.append(pl.BlockSpec((1, f), lambda i: (0, 0)))
            args.append(p)
    out_shapes = [jax.ShapeDtypeStruct((npad, fout), jnp.float32)]
    out_specs = [pl.BlockSpec((BN, fout), lambda i: (i, 0))]
    if stats:
        out_shapes += [jax.ShapeDtypeStruct((1, fout), jnp.float32)] * 2
        out_specs += [pl.BlockSpec((1, fout), lambda i: (0, 0))] * 2

    res = pl.pallas_call(
        body,
        grid=(npad // BN,),
        in_specs=in_specs,
        out_specs=out_specs,
        out_shape=out_shapes,
        scratch_shapes=[pltpu.VMEM((V, 6 * fout), jnp.float32)],
    )(*args)
    if stats:
        return res[0], res[1], res[2]
    return res[0]


def _conv(table, no_idx, W, b, npad, n, pro, stats=True):
    no6_pad = no_idx[0] if isinstance(no_idx, tuple) else no_idx
    V, f = table.shape
    fout = W.shape[1]
    unfused = 12.0 * npad * V * f
    fused = 12.0 * npad * V * fout + 12.0 * V * f * fout
    pro6 = _pro2d(pro, 6) if pro is not None else None
    pro1 = _pro2d(pro) if pro is not None else None
    if min(unfused, fused) <= _TC_GATHER_FLOPS:
        if fused <= unfused * 1.25 + 2e9:
            return _tc_conv_fused(table, no6_pad, W, b, npad, n, pro, stats)
        parts = [
            (_tc_gather6(table, no6_pad, npad), W[: 6 * f], pro6),
            (table, W[6 * f :], pro1),
        ]
    else:
        parts = [
            (_sc_gather(table, no6_pad).reshape(npad, 6 * f), W[: 6 * f], pro6),
            (table, W[6 * f :], pro1),
        ]
    return _tc_matmul(parts, b, nvalid=n) if stats else _tc_matmul(parts, b)


def _double_conv(table, n, npad, no6_pad, p, pro_in):
    y1, s1, ss1 = _conv(table, no6_pad, p["c1"]["W"], p["c1"]["b"], npad, n, pro_in)
    pro1 = _bn_finalize(s1, ss1, n, p["bn1"])
    y2, s2, ss2 = _conv(y1, no6_pad, p["c2"]["W"], p["c2"]["b"], npad, n, pro1)
    pro2 = _bn_finalize(s2, ss2, n, p["bn2"])
    return y2, pro2


def kernel(x, params, idx):
    levels = _LEVELS
    npads = [_rup(n, 512) for n in levels]
    no6_pad = []
    for i, n in enumerate(levels):
        no2d = idx["neigh_%d" % n].reshape(n, 7)[:, :6]
        no6_pad.append(_pad_idx(no2d.reshape(-1), npads[i] * 6))

    x16 = jnp.pad(x, ((0, npads[0] - levels[0]), (0, 13)))
    W1 = params["down1"]["c1"]["W"].reshape(7, 3, -1)
    W1p = jnp.zeros((7, 16, W1.shape[2]), jnp.float32).at[:, :3, :].set(W1)
    W1p = W1p.reshape(7 * 16, -1)
    p1 = {
        "c1": {"W": W1p, "b": params["down1"]["c1"]["b"]},
        "bn1": params["down1"]["bn1"],
        "c2": params["down1"]["c2"],
        "bn2": params["down1"]["bn2"],
    }

    skips = []
    table, pro_in = x16, None
    for i in range(5):
        n, npad = levels[i], npads[i]
        p = p1 if i == 0 else params["down%d" % (i + 1)]
        y, pro = _double_conv(table, n, npad, no6_pad[i], p, pro_in)
        if i < 4:
            skips.append((y, pro))
            r, rpad = levels[i + 1], npads[i + 1]
            f = y.shape[1]
            pno6 = _pad_idx(
                _drop_self(idx["neigh_%d" % n][: r * 7], r), rpad * 6
            )
            table = _conv(
                y, pno6, _pool_matrix(f), jnp.zeros((f,), jnp.float32),
                rpad, 0, pro, stats=False,
            )
            pro_in = None
        else:
            h, pro_h = y, pro

    for j in range(1, 5):
        lev = 4 - j
        n, npad = levels[lev], npads[lev]
        r = (n + 6) // 4
        p = params["up%d" % j]
        fout = p["c1"]["W"].shape[1]

        y_up = _tc_matmul(
            [(h, p["up"]["W"], _pro2d(pro_h))], p["up"]["b"]
        )
        y_rows = y_up.reshape(-1, fout)

        top = idx["up_top_%d" % n]
        down = idx["up_down_%d" % n]
        Bp_t = _rup(top.shape[0], 8 * _NW)
        Bp_d = _rup(down.shape[0], 8 * _NW)
        idx_td = jnp.concatenate([_pad_idx(top, Bp_t), _pad_idx(down, Bp_d)])
        Vr = y_rows.shape[0]
        if 2.0 * idx_td.shape[0] * Vr * fout <= _TC_GATHER_FLOPS:
            G_td = _tc_gather1(y_rows, idx_td)
        else:
            G_td = _sc_gather(y_rows, idx_td)
        x1 = G_td[:r]
        Gd = G_td[Bp_t:].reshape(Bp_d // 2, 2 * fout)
        x2 = _tc_matmul(
            [(Gd, _updown_matrix(fout), None)], jnp.zeros((fout,), jnp.float32)
        )
        h_up = jnp.concatenate([x1, x2[: n - r]], axis=0)

        y_skip, pro_skip = skips[lev]
        hc = jnp.concatenate([h_up, y_skip[:n]], axis=1)
        hc = jnp.pad(hc, ((0, npad - n), (0, 0)))
        ones = jnp.ones((fout,), jnp.float32)
        zeros = jnp.zeros((fout,), jnp.float32)
        pro_c = (
            jnp.concatenate([ones, pro_skip[0]]),
            jnp.concatenate([zeros, pro_skip[1]]),
            jnp.concatenate([ones, pro_skip[2]]),
        )
        h, pro_h = _double_conv(hc, n, npad, no6_pad[lev], p, pro_c)

    out = _tc_matmul(
        [(h, params["outc"]["W"], _pro2d(pro_h))], params["outc"]["b"]
    )
    return out[: levels[0]]


# speedup vs baseline: 1.0271x; 1.0062x over previous
import functools

import numpy as np

import jax
import jax.numpy as jnp
from jax import lax
from jax.experimental import pallas as pl
from jax.experimental.pallas import tpu as pltpu
from jax.experimental.pallas import tpu_sc as plsc

_LEVELS = [40962, 10242, 2562, 642, 162]
_EPS = 1e-5

_NC = 2
_NS = 16
_NW = _NC * _NS


def _rup(x, m):
    return (x + m - 1) // m * m


@functools.lru_cache(maxsize=None)
def _make_sc_gather(V, D, Bp):
    assert Bp % (8 * _NW) == 0
    b_per_w = Bp // _NW
    C = min(128 if D <= 256 else 64, b_per_w)
    NBUF = max(1, min(8, 393216 // (C * D * 4)))
    nfull = b_per_w // C
    tail = b_per_w % C
    ngrp = nfull // NBUF
    nrem = nfull % NBUF

    mesh = plsc.VectorSubcoreMesh(core_axis_name="c", subcore_axis_name="s")
    scratch = [pltpu.VMEM((b_per_w,), jnp.int32)]
    scratch += [pltpu.VMEM((C, D), jnp.float32) for _ in range(NBUF)]
    scratch += [pltpu.SemaphoreType.DMA for _ in range(2 * NBUF)]

    @functools.partial(
        pl.kernel,
        mesh=mesh,
        out_type=jax.ShapeDtypeStruct((Bp, D), jnp.float32),
        compiler_params=pltpu.CompilerParams(use_tc_tiling_on_sc=False),
        scratch_types=scratch,
    )
    def gather_kernel(table_hbm, idx_hbm, out_hbm, idx_v, *rest):
        bufs = rest[:NBUF]
        gsem = rest[NBUF : 2 * NBUF]
        wsem = rest[2 * NBUF : 3 * NBUF]
        wid = lax.axis_index("s") * _NC + lax.axis_index("c")
        base = wid * b_per_w
        pltpu.sync_copy(idx_hbm.at[pl.ds(base, b_per_w)], idx_v)

        def fire_gather(off, b):
            pltpu.async_copy(
                table_hbm.at[idx_v.at[pl.ds(off, C)]], bufs[b], gsem[b]
            )

        def wait_gather(b):
            pltpu.make_async_copy(
                table_hbm.at[idx_v.at[pl.ds(0, C)]], bufs[b], gsem[b]
            ).wait()

        def fire_wb(off, b):
            pltpu.async_copy(bufs[b], out_hbm.at[pl.ds(base + off, C)], wsem[b])

        def wait_wb(b):
            pltpu.make_async_copy(bufs[b], out_hbm.at[pl.ds(0, C)], wsem[b]).wait()

        def group(g, carry):
            for b in range(NBUF):
                c = g * NBUF + b

                @pl.when(g > 0)
                def _(b=b):
                    wait_wb(b)

                fire_gather(c * C, b)
                if b > 0:
                    wait_gather(b - 1)
                    fire_wb((c - 1) * C, b - 1)
                else:

                    @pl.when(g > 0)
                    def _(c=c):
                        wait_gather(NBUF - 1)
                        fire_wb((c - 1) * C, NBUF - 1)

            return carry

        if ngrp > 0:
            lax.fori_loop(0, ngrp, group, 0)
            wait_gather(NBUF - 1)
            fire_wb((ngrp * NBUF - 1) * C, NBUF - 1)
            for b in range(NBUF):
                wait_wb(b)

        off0 = ngrp * NBUF * C
        for j in range(nrem):
            off = off0 + j * C
            pltpu.async_copy(
                table_hbm.at[idx_v.at[pl.ds(off, C)]], bufs[0], gsem[0]
            ).wait()
            pltpu.sync_copy(bufs[0], out_hbm.at[pl.ds(base + off, C)])
        if tail:
            off = nfull * C
            pltpu.async_copy(
                table_hbm.at[idx_v.at[pl.ds(off, tail)]],
                bufs[0].at[pl.ds(0, tail)],
                gsem[0],
            ).wait()
            pltpu.sync_copy(
                bufs[0].at[pl.ds(0, tail)], out_hbm.at[pl.ds(base + off, tail)]
            )

    return gather_kernel


def _sc_gather(table, idxp):
    V, D = table.shape
    (Bp,) = idxp.shape
    return _make_sc_gather(V, D, Bp)(table, idxp)


def _tc_gather6(table, no6p, npad):
    V, f = table.shape
    idx2 = no6p.reshape(npad, 6)
    BN = _pick_bn(npad)

    def body(idx_ref, t_ref, out_ref):
        iota = lax.broadcasted_iota(jnp.int32, (BN, V), 1)
        t = t_ref[...]
        for k in range(6):
            sel = idx_ref[:, k : k + 1]
            M = (iota == sel).astype(jnp.float32)
            out_ref[:, k * f : (k + 1) * f] = jnp.dot(
                M, t, preferred_element_type=jnp.float32
            )

    return pl.pallas_call(
        body,
        grid=(npad // BN,),
        in_specs=[
            pl.BlockSpec((BN, 6), lambda i: (i, 0)),
            pl.BlockSpec((V, f), lambda i: (0, 0)),
        ],
        out_specs=pl.BlockSpec((BN, 6 * f), lambda i: (i, 0)),
        out_shape=jax.ShapeDtypeStruct((npad, 6 * f), jnp.float32),
    )(idx2, table)


def _tc_gather_mean(table, ia, ib):
    V, f = table.shape
    (Bp,) = ia.shape
    BN = _pick_bn(Bp)

    def body(a_ref, b_ref, t_ref, out_ref):
        iota = lax.broadcasted_iota(jnp.int32, (BN, V), 1)
        M = (
            (iota == a_ref[...]).astype(jnp.float32)
            + (iota == b_ref[...]).astype(jnp.float32)
        ) * 0.5
        out_ref[...] = jnp.dot(M, t_ref[...], preferred_element_type=jnp.float32)

    return pl.pallas_call(
        body,
        grid=(Bp // BN,),
        in_specs=[
            pl.BlockSpec((BN, 1), lambda i: (i, 0)),
            pl.BlockSpec((BN, 1), lambda i: (i, 0)),
            pl.BlockSpec((V, f), lambda i: (0, 0)),
        ],
        out_specs=pl.BlockSpec((BN, f), lambda i: (i, 0)),
        out_shape=jax.ShapeDtypeStruct((Bp, f), jnp.float32),
    )(ia.reshape(Bp, 1), ib.reshape(Bp, 1), table)


def _tc_gather1(table, idxp):
    V, f = table.shape
    (Bp,) = idxp.shape
    BN = _pick_bn(Bp)

    def body(idx_ref, t_ref, out_ref):
        iota = lax.broadcasted_iota(jnp.int32, (BN, V), 1)
        M = (iota == idx_ref[...]).astype(jnp.float32)
        out_ref[...] = jnp.dot(M, t_ref[...], preferred_element_type=jnp.float32)

    return pl.pallas_call(
        body,
        grid=(Bp // BN,),
        in_specs=[
            pl.BlockSpec((BN, 1), lambda i: (i, 0)),
            pl.BlockSpec((V, f), lambda i: (0, 0)),
        ],
        out_specs=pl.BlockSpec((BN, f), lambda i: (i, 0)),
        out_shape=jax.ShapeDtypeStruct((Bp, f), jnp.float32),
    )(idxp.reshape(Bp, 1), table)


def _pick_bn(M):
    for b in (512, 256, 128, 64, 32, 16, 8):
        if M % b == 0:
            return b
    raise ValueError(M)


def _tc_matmul(parts, bias, nvalid=None):
    M = parts[0][0].shape[0]
    F = parts[0][1].shape[1]
    BN = _pick_bn(M)
    stats = nvalid is not None
    meta = tuple(p[2] is not None for p in parts)

    def body(*refs):
        i = pl.program_id(0)
        it = iter(refs)
        acc = None
        for has_pro in meta:
            x = next(it)[...]
            w = next(it)[...]
            if has_pro:
                sc, sh, sl = next(it)[...], next(it)[...], next(it)[...]
                v = x * sc + sh
                x = jnp.maximum(v, 0.0) + sl * jnp.minimum(v, 0.0)
            d = jnp.dot(x, w, preferred_element_type=jnp.float32)
            acc = d if acc is None else acc + d
        y = acc + next(it)[...]
        y_ref = next(it)
        y_ref[...] = y
        if stats:
            s_ref = next(it)
            ss_ref = next(it)
            rows = lax.broadcasted_iota(jnp.int32, (BN, 1), 0) + i * BN
            m = (rows < nvalid).astype(jnp.float32)
            ym = y * m
            ps = jnp.sum(ym, axis=0, keepdims=True)
            pss = jnp.sum(ym * ym, axis=0, keepdims=True)

            @pl.when(i == 0)
            def _():
                s_ref[...] = jnp.zeros_like(s_ref)
                ss_ref[...] = jnp.zeros_like(ss_ref)

            s_ref[...] += ps
            ss_ref[...] += pss

    in_specs = []
    args = []
    for X, W, pro in parts:
        K = X.shape[1]
        in_specs.append(pl.BlockSpec((BN, K), lambda i: (i, 0)))
        in_specs.append(pl.BlockSpec((K, F), lambda i: (0, 0)))
        args += [X, W]
        if pro is not None:
            for p in pro:
                in_specs.append(pl.BlockSpec((1, K), lambda i: (0, 0)))
                args.append(p)
    in_specs.append(pl.BlockSpec((1, F), lambda i: (0, 0)))
    args.append(bias.reshape(1, F))

    out_shapes = [jax.ShapeDtypeStruct((M, F), jnp.float32)]
    out_specs = [pl.BlockSpec((BN, F), lambda i: (i, 0))]
    if stats:
        out_shapes += [jax.ShapeDtypeStruct((1, F), jnp.float32)] * 2
        out_specs += [pl.BlockSpec((1, F), lambda i: (0, 0))] * 2

    res = pl.pallas_call(
        body,
        grid=(M // BN,),
        in_specs=in_specs,
        out_specs=out_specs,
        out_shape=out_shapes,
    )(*args)
    if stats:
        return res[0], res[1], res[2]
    return res[0]


def _bn_finalize(s, ss, n, bnp):
    mu = s[0] / n
    var = jnp.maximum(ss[0] / n - mu * mu, 0.0)
    rstd = lax.rsqrt(var + _EPS)
    scale = bnp["g"] * rstd
    shift = bnp["b"] - mu * scale
    slope = jnp.full_like(scale, 0.2)
    return scale, shift, slope


def _pro2d(pro, reps=1):
    return tuple(jnp.tile(p, reps).reshape(1, -1) for p in pro)


def _pad_idx(a, Bp):
    B = a.shape[0]
    return jnp.pad(a, (0, Bp - B)) if Bp != B else a


def _drop_self(no, n):
    return no.reshape(n, 7)[:, :6].reshape(-1)


def _pool_matrix(f):
    P = np.zeros((7 * f, f), np.float32)
    P[np.arange(7 * f), np.arange(7 * f) // 7] = 1.0 / 7.0
    return jnp.asarray(P)


def _updown_matrix(f):
    Q = np.zeros((2 * f, f), np.float32)
    Q[np.arange(2 * f), np.arange(2 * f) // 2] = 0.5
    return jnp.asarray(Q)


_TC_GATHER_FLOPS = 3e10


def _tc_conv_fused(table, no6p, W, bias, npad, n, pro, stats):
    V, f = table.shape
    fout = W.shape[1]
    idx2 = no6p.reshape(npad, 6)
    BN = _pick_bn(npad)
    W6n = W[: 6 * f].reshape(6, f, fout).transpose(1, 0, 2).reshape(f, 6 * fout)
    Wself = W[6 * f :]
    has_pro = pro is not None

    def body(*refs):
        it = iter(refs)
        idx_ref = next(it)
        tfull_ref = next(it)
        tblk_ref = next(it)
        w6_ref = next(it)
        ws_ref = next(it)
        b_ref = next(it)
        if has_pro:
            sc_ref, sh_ref, sl_ref = next(it), next(it), next(it)
        y_ref = next(it)
        if stats:
            s_ref, ss_ref = next(it), next(it)
        T_ref = next(it)
        i = pl.program_id(0)

        def act(v):
            if not has_pro:
                return v
            u = v * sc_ref[...] + sh_ref[...]
            return jnp.maximum(u, 0.0) + sl_ref[...] * jnp.minimum(u, 0.0)

        @pl.when(i == 0)
        def _():
            T_ref[...] = jnp.dot(
                act(tfull_ref[...]), w6_ref[...],
                preferred_element_type=jnp.float32,
            )

        iota = lax.broadcasted_iota(jnp.int32, (BN, V), 1)
        acc = jnp.dot(
            act(tblk_ref[...]), ws_ref[...], preferred_element_type=jnp.float32
        )
        for k in range(6):
            M = (iota == idx_ref[:, k : k + 1]).astype(jnp.float32)
            acc = acc + jnp.dot(
                M,
                T_ref[:, k * fout : (k + 1) * fout],
                preferred_element_type=jnp.float32,
            )
        y = acc + b_ref[...]
        y_ref[...] = y
        if stats:
            rows = lax.broadcasted_iota(jnp.int32, (BN, 1), 0) + i * BN
            m = (rows < n).astype(jnp.float32)
            ym = y * m
            ps = jnp.sum(ym, axis=0, keepdims=True)
            pss = jnp.sum(ym * ym, axis=0, keepdims=True)

            @pl.when(i == 0)
            def _():
                s_ref[...] = jnp.zeros_like(s_ref)
                ss_ref[...] = jnp.zeros_like(ss_ref)

            s_ref[...] += ps
            ss_ref[...] += pss

    in_specs = [
        pl.BlockSpec((BN, 6), lambda i: (i, 0)),
        pl.BlockSpec((V, f), lambda i: (0, 0)),
        pl.BlockSpec((BN, f), lambda i: (i, 0)),
        pl.BlockSpec((f, 6 * fout), lambda i: (0, 0)),
        pl.BlockSpec((f, fout), lambda i: (0, 0)),
        pl.BlockSpec((1, fout), lambda i: (0, 0)),
    ]
    args = [idx2, table, table, W6n, Wself, bias.reshape(1, fout)]
    if has_pro:
        for p in _pro2d(pro):
            in_specs.append(pl.BlockSpec((1, f), lambda i: (0, 0)))
            args.append(p)
    out_shapes = [jax.ShapeDtypeStruct((npad, fout), jnp.float32)]
    out_specs = [pl.BlockSpec((BN, fout), lambda i: (i, 0))]
    if stats:
        out_shapes += [jax.ShapeDtypeStruct((1, fout), jnp.float32)] * 2
        out_specs += [pl.BlockSpec((1, fout), lambda i: (0, 0))] * 2

    res = pl.pallas_call(
        body,
        grid=(npad // BN,),
        in_specs=in_specs,
        out_specs=out_specs,
        out_shape=out_shapes,
        scratch_shapes=[pltpu.VMEM((V, 6 * fout), jnp.float32)],
    )(*args)
    if stats:
        return res[0], res[1], res[2]
    return res[0]


def _conv(table, no_idx, W, b, npad, n, pro, stats=True):
    no6_pad = no_idx[0] if isinstance(no_idx, tuple) else no_idx
    V, f = table.shape
    fout = W.shape[1]
    unfused = 12.0 * npad * V * f
    fused = 12.0 * npad * V * fout + 12.0 * V * f * fout
    pro6 = _pro2d(pro, 6) if pro is not None else None
    pro1 = _pro2d(pro) if pro is not None else None
    if min(unfused, fused) <= _TC_GATHER_FLOPS:
        if fused <= unfused * 1.25 + 2e9:
            return _tc_conv_fused(table, no6_pad, W, b, npad, n, pro, stats)
        parts = [
            (_tc_gather6(table, no6_pad, npad), W[: 6 * f], pro6),
            (table, W[6 * f :], pro1),
        ]
    else:
        parts = [
            (_sc_gather(table, no6_pad).reshape(npad, 6 * f), W[: 6 * f], pro6),
            (table, W[6 * f :], pro1),
        ]
    return _tc_matmul(parts, b, nvalid=n) if stats else _tc_matmul(parts, b)


def _double_conv(table, n, npad, no6_pad, p, pro_in):
    y1, s1, ss1 = _conv(table, no6_pad, p["c1"]["W"], p["c1"]["b"], npad, n, pro_in)
    pro1 = _bn_finalize(s1, ss1, n, p["bn1"])
    y2, s2, ss2 = _conv(y1, no6_pad, p["c2"]["W"], p["c2"]["b"], npad, n, pro1)
    pro2 = _bn_finalize(s2, ss2, n, p["bn2"])
    return y2, pro2


def kernel(x, params, idx):
    levels = _LEVELS
    npads = [_rup(n, 512) for n in levels]
    no6_pad = []
    for i, n in enumerate(levels):
        no2d = idx["neigh_%d" % n].reshape(n, 7)[:, :6]
        no6_pad.append(_pad_idx(no2d.reshape(-1), npads[i] * 6))

    x16 = jnp.pad(x, ((0, npads[0] - levels[0]), (0, 13)))
    W1 = params["down1"]["c1"]["W"].reshape(7, 3, -1)
    W1p = jnp.zeros((7, 16, W1.shape[2]), jnp.float32).at[:, :3, :].set(W1)
    W1p = W1p.reshape(7 * 16, -1)
    p1 = {
        "c1": {"W": W1p, "b": params["down1"]["c1"]["b"]},
        "bn1": params["down1"]["bn1"],
        "c2": params["down1"]["c2"],
        "bn2": params["down1"]["bn2"],
    }

    skips = []
    table, pro_in = x16, None
    for i in range(5):
        n, npad = levels[i], npads[i]
        p = p1 if i == 0 else params["down%d" % (i + 1)]
        y, pro = _double_conv(table, n, npad, no6_pad[i], p, pro_in)
        if i < 4:
            skips.append((y, pro))
            r, rpad = levels[i + 1], npads[i + 1]
            f = y.shape[1]
            pno6 = _pad_idx(
                _drop_self(idx["neigh_%d" % n][: r * 7], r), rpad * 6
            )
            table = _conv(
                y, pno6, _pool_matrix(f), jnp.zeros((f,), jnp.float32),
                rpad, 0, pro, stats=False,
            )
            pro_in = None
        else:
            h, pro_h = y, pro

    for j in range(1, 5):
        lev = 4 - j
        n, npad = levels[lev], npads[lev]
        r = (n + 6) // 4
        p = params["up%d" % j]
        fout = p["c1"]["W"].shape[1]

        y_up = _tc_matmul(
            [(h, p["up"]["W"], _pro2d(pro_h))], p["up"]["b"]
        )
        y_rows = y_up.reshape(-1, fout)

        top = idx["up_top_%d" % n]
        down = idx["up_down_%d" % n]
        Vr = y_rows.shape[0]
        y_skip, pro_skip = skips[lev]
        if 4.0 * npad * Vr * fout <= _TC_GATHER_FLOPS and npad * Vr <= 3e7:
            A = _pad_idx(jnp.concatenate([top, down[0::2]]), npad)
            Bc = _pad_idx(jnp.concatenate([top, down[1::2]]), npad)
            h_up = _tc_gather_mean(y_rows, A, Bc)
            hc = jnp.concatenate([h_up, y_skip], axis=1)
        else:
            Bp_t = _rup(top.shape[0], 8 * _NW)
            Bp_d = _rup(down.shape[0], 8 * _NW)
            idx_td = jnp.concatenate([_pad_idx(top, Bp_t), _pad_idx(down, Bp_d)])
            G_td = _sc_gather(y_rows, idx_td)
            x1 = G_td[:r]
            Gd = G_td[Bp_t:].reshape(Bp_d // 2, 2 * fout)
            x2 = _tc_matmul(
                [(Gd, _updown_matrix(fout), None)],
                jnp.zeros((fout,), jnp.float32),
            )
            h_up = jnp.concatenate([x1, x2[: n - r]], axis=0)
            hc = jnp.concatenate([h_up, y_skip[:n]], axis=1)
            hc = jnp.pad(hc, ((0, npad - n), (0, 0)))
        ones = jnp.ones((fout,), jnp.float32)
        zeros = jnp.zeros((fout,), jnp.float32)
        pro_c = (
            jnp.concatenate([ones, pro_skip[0]]),
            jnp.concatenate([zeros, pro_skip[1]]),
            jnp.concatenate([ones, pro_skip[2]]),
        )
        h, pro_h = _double_conv(hc, n, npad, no6_pad[lev], p, pro_c)

    out = _tc_matmul(
        [(h, params["outc"]["W"], _pro2d(pro_h))], params["outc"]["b"]
    )
    return out[: levels[0]]
